# baseline (speedup 1.0000x reference)
import functools

import jax
import jax.numpy as jnp
from jax import lax
from jax.experimental import pallas as pl
from jax.experimental.pallas import tpu as pltpu
from jax.experimental.pallas import tpu_sc as plsc

_B = 16384
_T = 50
_BATCH = _B * _T
_DIM = 64
_NW = 32
_BPW = _B // _NW
_IDXW = _BPW * _T
_SUB = 256
_NSUB = _BPW // _SUB
_NSTEP = _T * _NSUB
_NBUF = 2


@functools.partial(
    pl.kernel,
    mesh=plsc.VectorSubcoreMesh(core_axis_name="c", subcore_axis_name="s"),
    out_type=jax.ShapeDtypeStruct((_T, 8, _B // 128, 8, 128), jnp.float32),
    scratch_types=[
        pltpu.VMEM((_IDXW,), jnp.int32),
        pltpu.VMEM((_IDXW,), jnp.int32),
        pltpu.VMEM((_NBUF, _SUB, _DIM), jnp.float32),
        pltpu.VMEM((_NBUF, 8, _SUB // 128, 8, 128), jnp.float32),
    ] + [pltpu.SemaphoreType.DMA] * (2 * _NBUF),
    compiler_params=pltpu.CompilerParams(
        use_tc_tiling_on_sc=False, needs_layout_passes=False
    ),
)
def _embed(idx_hbm, table_hbm, out_hbm, raw_v, idxt_v, rows_v, x5_v, *sems):
    gs = sems[:_NBUF]
    ss = sems[_NBUF:]
    wid = lax.axis_index("s") * 2 + lax.axis_index("c")
    b0 = wid * _BPW
    jc = wid * (_BPW // 128)
    iota = lax.iota(jnp.int32, 16)
    iota64 = iota * _DIM

    pltpu.sync_copy(idx_hbm.at[pl.ds(b0 * _T, _IDXW)], raw_v)

    def tbody(t, c):
        def bbody(g, c2):
            src = plsc.load_gather(raw_v, [(g * 16 + iota) * _T + t])
            idxt_v[pl.ds(t * _BPW + g * 16, 16)] = src
            return c2

        return lax.fori_loop(0, _BPW // 16, bbody, c, unroll=4)

    lax.fori_loop(0, _T, tbody, 0)

    def start_gather(step, buf):
        off = pl.multiple_of(step * _SUB, 8)
        pltpu.async_copy(
            table_hbm.at[idxt_v.at[pl.ds(off, _SUB)]],
            rows_v.at[buf],
            gs[buf],
        )

    def wait_gather(buf):
        pltpu.make_async_copy(
            table_hbm.at[pl.ds(0, _SUB)],
            rows_v.at[buf],
            gs[buf],
        ).wait()

    def start_store(step, buf):
        t = step >> 1
        sub = step & 1
        pltpu.async_copy(
            x5_v.at[buf],
            out_hbm.at[t, :, pl.ds(jc + sub * (_SUB // 128), _SUB // 128)],
            ss[buf],
        )

    def wait_store(buf):
        pltpu.make_async_copy(
            x5_v.at[buf],
            out_hbm.at[0, :, pl.ds(0, _SUB // 128)],
            ss[buf],
        ).wait()

    def transpose_sub(buf):
        rows = rows_v.at[buf]
        x5 = x5_v.at[buf]

        def dbody(d, c):
            i = lax.shift_right_logical(d, 3)
            s = lax.bitwise_and(d, 7)
            dsplat = jnp.zeros((16,), jnp.int32) + d
            for g in range(_SUB // 16):
                j0 = g * 16
                v = plsc.load_gather(rows, [j0 + iota, dsplat])
                x5[i, j0 // 128, s, pl.ds(j0 % 128, 16)] = v
            return c

        lax.fori_loop(0, _DIM, dbody, 0, unroll=4)

    start_gather(0, 0)
    start_gather(1, 1)

    def outer(g, c):
        for b in range(_NBUF):
            step = g * _NBUF + b
            wait_gather(b)

            @pl.when(g > 0)
            def _():
                wait_store(b)

            transpose_sub(b)
            start_store(step, b)

            @pl.when(step + _NBUF < _NSTEP)
            def _():
                start_gather(step + _NBUF, b)

        return c

    lax.fori_loop(0, _NSTEP // _NBUF, outer, 0)
    for b in range(_NBUF):
        wait_store(b)


def kernel(token_ids, weight):
    idx = jnp.reshape(token_ids.astype(jnp.int32), (_BATCH,))
    x5 = _embed(idx, weight)
    out = jnp.transpose(x5, (2, 4, 0, 1, 3))
    return jnp.reshape(out, (_B, _T, _DIM))


# speedup vs baseline: 1.4619x; 1.4619x over previous
import functools

import jax
import jax.numpy as jnp
from jax import lax
from jax.experimental import pallas as pl
from jax.experimental.pallas import tpu as pltpu
from jax.experimental.pallas import tpu_sc as plsc

_NUM_ROWS = 1000000
_DIM = 64
_BATCH = 16384 * 50
_NUM_WORKERS = 32
_B_PER_W = _BATCH // _NUM_WORKERS
_NBUF = 4
_CHUNK = 320
_NCHUNKS = _B_PER_W // _CHUNK
_NOUTER = _NCHUNKS // _NBUF


@functools.partial(
    pl.kernel,
    mesh=plsc.VectorSubcoreMesh(core_axis_name="c", subcore_axis_name="s"),
    out_type=jax.ShapeDtypeStruct((_BATCH, _DIM), jnp.float32),
    scratch_types=[
        pltpu.VMEM((_B_PER_W,), jnp.int32),
        pltpu.VMEM((_NBUF, _CHUNK, _DIM), jnp.float32),
    ] + [pltpu.SemaphoreType.DMA] * (2 * _NBUF),
    compiler_params=pltpu.CompilerParams(use_tc_tiling_on_sc=False),
)
def _embed_gather(idx_hbm, table_hbm, out_hbm, idx_v, rows_v, *sems):
    gsems = sems[:_NBUF]
    ssems = sems[_NBUF:]
    wid = lax.axis_index("s") * 2 + lax.axis_index("c")
    base = wid * _B_PER_W
    pltpu.sync_copy(idx_hbm.at[pl.ds(base, _B_PER_W)], idx_v)

    def start_gather(chunk, b):
        off = pl.multiple_of(chunk * _CHUNK, 8)
        pltpu.async_copy(
            table_hbm.at[idx_v.at[pl.ds(off, _CHUNK)]], rows_v.at[b], gsems[b]
        )

    def wait_gather(b):
        pltpu.make_async_copy(
            table_hbm.at[pl.ds(0, _CHUNK)], rows_v.at[b], gsems[b]
        ).wait()

    def start_store(chunk, b):
        off = pl.multiple_of(base + chunk * _CHUNK, 8)
        pltpu.async_copy(rows_v.at[b], out_hbm.at[pl.ds(off, _CHUNK)], ssems[b])

    def wait_store(b):
        pltpu.make_async_copy(
            rows_v.at[b], out_hbm.at[pl.ds(0, _CHUNK)], ssems[b]
        ).wait()

    for b in range(_NBUF):
        start_gather(b, b)

    def outer(g, carry):
        cbase = g * _NBUF
        for b in range(_NBUF):
            wait_gather(b)
            start_store(cbase + b, b)
        for b in range(_NBUF):

            @pl.when(g < _NOUTER - 1)
            def _():
                wait_store(b)
                start_gather(cbase + _NBUF + b, b)

        return carry

    lax.fori_loop(0, _NOUTER, outer, 0)
    for b in range(_NBUF):
        wait_store(b)


def kernel(token_ids, weight):
    idx = jnp.reshape(token_ids.astype(jnp.int32), (_BATCH,))
    out = _embed_gather(idx, weight)
    return jnp.reshape(out, (*token_ids.shape, _DIM))
